# baseline (speedup 1.0000x reference)
import functools

import jax
import jax.numpy as jnp
from jax.experimental import pallas as pl
from jax.experimental.pallas import tpu as pltpu

_EMBED = 64
_HEADS = 4
_MODEL_CAT_RATE = 0.02
_ID_CAT_RATE = 0.36


def _dot(g, x, prec):
    if prec == "f32":
        return jnp.dot(g, x, preferred_element_type=jnp.float32)
    g_b = g.astype(jnp.bfloat16)
    x_hi = x.astype(jnp.bfloat16)
    if prec == "bf16":
        return jnp.dot(g_b, x_hi, preferred_element_type=jnp.float32)
    x_lo = (x - x_hi.astype(jnp.float32)).astype(jnp.bfloat16)
    return (jnp.dot(g_b, x_hi, preferred_element_type=jnp.float32)
            + jnp.dot(g_b, x_lo, preferred_element_type=jnp.float32))


def _mm_body(g_ref, x_ref, o_ref, *, softmax, prec):
    acc = _dot(g_ref[...], x_ref[...], prec)
    if softmax:
        acc = jax.nn.softmax(acc, axis=-1)
    o_ref[...] = acc


def _mm_bias_body(g_ref, x_ref, b_ref, o_ref, *, prec):
    acc = _dot(g_ref[...], x_ref[...], prec)
    o_ref[...] = acc + b_ref[...]


def _rowmm(g, x, bias=None, softmax=False, bm=1024, prec="f32"):
    m, k = g.shape
    c = x.shape[1]
    in_specs = [
        pl.BlockSpec((bm, k), lambda i: (i, 0)),
        pl.BlockSpec((k, c), lambda i: (0, 0)),
    ]
    args = [g, x]
    if bias is None:
        body = functools.partial(_mm_body, softmax=softmax, prec=prec)
    else:
        body = functools.partial(_mm_bias_body, prec=prec)
        in_specs.append(pl.BlockSpec((1, c), lambda i: (0, 0)))
        args.append(bias.reshape(1, c))
    return pl.pallas_call(
        body,
        grid=(m // bm,),
        in_specs=in_specs,
        out_specs=pl.BlockSpec((bm, c), lambda i: (i, 0)),
        out_shape=jax.ShapeDtypeStruct((m, c), jnp.float32),
        compiler_params=pltpu.CompilerParams(
            dimension_semantics=("parallel",)),
    )(*args)


def _multi_mm_body(*refs):
    n = len(refs) - 3
    g = pl.program_id(0)
    x_ref, b_ref, o_ref = refs[n], refs[n + 1], refs[n + 2]
    for j in range(n):
        @pl.when(g == j)
        def _(j=j):
            o_ref[0] = (jnp.dot(refs[j][...], x_ref[0],
                                preferred_element_type=jnp.float32)
                        + b_ref[0])


def _multi_mm(graphs, rhs, biases, bm=512):
    n = len(graphs)
    m, k = graphs[0].shape
    c = rhs[0].shape[1]
    nb = m // bm
    x = jnp.stack(rhs)
    b = jnp.stack([jnp.zeros((1, c), jnp.float32) if bb is None
                   else bb.reshape(1, c) for bb in biases])

    def g_index(j):
        def idx(g, i):
            return (jnp.where(g < j, 0, jnp.where(g > j, nb - 1, i)), 0)
        return idx

    in_specs = [pl.BlockSpec((bm, k), g_index(j)) for j in range(n)]
    in_specs.append(pl.BlockSpec((1, k, c), lambda g, i: (g, 0, 0)))
    in_specs.append(pl.BlockSpec((1, 1, c), lambda g, i: (g, 0, 0)))
    out = pl.pallas_call(
        _multi_mm_body,
        grid=(n, nb),
        in_specs=in_specs,
        out_specs=pl.BlockSpec((1, bm, c), lambda g, i: (g, i, 0)),
        out_shape=jax.ShapeDtypeStruct((n, m, c), jnp.float32),
        compiler_params=pltpu.CompilerParams(
            dimension_semantics=("arbitrary", "arbitrary")),
    )(*graphs, x, b)
    return out


def _row_normalize(z):
    n = jnp.sqrt(jnp.sum(z * z, axis=1, keepdims=True))
    return z / jnp.maximum(n, 1e-12)


def _id_fuse_body(a_ref, b_ref, emb_ref, w_ref, o_ref):
    z = jnp.dot(0.5 * (a_ref[...] + b_ref[...]), w_ref[...],
                preferred_element_type=jnp.float32)
    o_ref[...] = emb_ref[...] + _ID_CAT_RATE * _row_normalize(z)


def _id_fuse(a, b, emb, w_sum, bm=512):
    m, c = a.shape
    spec = pl.BlockSpec((bm, c), lambda i: (i, 0))
    return pl.pallas_call(
        _id_fuse_body,
        grid=(m // bm,),
        in_specs=[spec, spec, spec, pl.BlockSpec((c, c), lambda i: (0, 0))],
        out_specs=spec,
        out_shape=jax.ShapeDtypeStruct((m, c), jnp.float32),
        compiler_params=pltpu.CompilerParams(
            dimension_semantics=("parallel",)),
    )(a, b, emb, w_sum)


def _final_body(g0_ref, g1_ref, g2_ref, fa_ref, fb_ref, o_ref):
    mean_g = (g0_ref[...] + g1_ref[...] + g2_ref[...]) * (1.0 / 3.0)
    o_ref[...] = (mean_g
                  + _MODEL_CAT_RATE * _row_normalize(fa_ref[...])
                  + _MODEL_CAT_RATE * _row_normalize(fb_ref[...]))


def _final_fuse(g0, g1, g2, fa, fb, bm=512):
    m, c = g0.shape
    spec = pl.BlockSpec((bm, c), lambda i: (i, 0))
    return pl.pallas_call(
        _final_body,
        grid=(m // bm,),
        in_specs=[spec] * 5,
        out_specs=spec,
        out_shape=jax.ShapeDtypeStruct((m, c), jnp.float32),
        compiler_params=pltpu.CompilerParams(
            dimension_semantics=("parallel",)),
    )(g0, g1, g2, fa, fb)


def kernel(ui_graph, iu_graph, image_ui_graph, image_iu_graph, text_ui_graph,
           text_iu_graph, image_feats, text_feats, w_image_trans, b_image_trans,
           w_text_trans, b_text_trans, user_id_emb, item_id_emb, w_q, w_k, w_cat):
    ma = _multi_mm(
        [image_feats, image_ui_graph, text_ui_graph],
        [w_image_trans, item_id_emb, item_id_emb],
        [b_image_trans, None, None])
    mb = _multi_mm(
        [image_iu_graph, text_iu_graph],
        [user_id_emb, user_id_emb],
        [None, None])
    image_f = ma[0]
    image_user_id = ma[1]
    text_user_id = ma[2]
    image_item_id = mb[0]
    text_item_id = mb[1]
    text_f = _rowmm(text_feats, w_text_trans, bias=b_text_trans)

    w_sum = w_cat.reshape(_HEADS, _EMBED, _EMBED).sum(0)
    u_g0 = _id_fuse(image_user_id, text_user_id, user_id_emb, w_sum)
    i_g0 = _id_fuse(image_item_id, text_item_id, item_id_emb, w_sum)

    xu = jnp.concatenate([image_f, text_f, i_g0], axis=1)
    u_cat = _rowmm(ui_graph, xu)
    i_cat = _rowmm(iu_graph, u_cat)
    image_user_feats = u_cat[:, :_EMBED]
    text_user_feats = u_cat[:, _EMBED:2 * _EMBED]
    u_g1 = u_cat[:, 2 * _EMBED:]
    image_item_feats = i_cat[:, :_EMBED]
    text_item_feats = i_cat[:, _EMBED:2 * _EMBED]
    i_g1 = i_cat[:, 2 * _EMBED:]

    u_g2 = _rowmm(ui_graph, i_g1, softmax=True)
    i_g2 = _rowmm(iu_graph, u_g2, softmax=True)

    u_g = _final_fuse(u_g0, u_g1, u_g2, image_user_feats, text_user_feats)
    i_g = _final_fuse(i_g0, i_g1, i_g2, image_item_feats, text_item_feats)

    return (u_g, i_g, image_item_feats, text_item_feats, image_user_feats,
            text_user_feats, u_g, i_g, image_user_id, text_user_id,
            image_item_id, text_item_id)


# speedup vs baseline: 1.1030x; 1.1030x over previous
import functools

import jax
import jax.numpy as jnp
from jax.experimental import pallas as pl
from jax.experimental.pallas import tpu as pltpu

_EMBED = 64
_HEADS = 4
_MODEL_CAT_RATE = 0.02
_ID_CAT_RATE = 0.36
_BM = 512
_F32 = jnp.float32


def _dot(a, b):
    return jnp.dot(a, b, preferred_element_type=_F32)


def _row_normalize(z):
    n = jnp.sqrt(jnp.sum(z * z, axis=1, keepdims=True))
    return z / jnp.maximum(n, 1e-12)


def _id_update(emb, a, b, w_sum):
    return emb + _ID_CAT_RATE * _row_normalize(_dot(0.5 * (a + b), w_sum))


def _mm_bias_body(g_ref, x_ref, b_ref, o_ref):
    o_ref[...] = _dot(g_ref[...], x_ref[...]) + b_ref[...]


def _proj(feats, w, b, bm=_BM):
    m, k = feats.shape
    c = w.shape[1]
    return pl.pallas_call(
        _mm_bias_body,
        grid=(m // bm,),
        in_specs=[pl.BlockSpec((bm, k), lambda i: (i, 0)),
                  pl.BlockSpec((k, c), lambda i: (0, 0)),
                  pl.BlockSpec((1, c), lambda i: (0, 0))],
        out_specs=pl.BlockSpec((bm, c), lambda i: (i, 0)),
        out_shape=jax.ShapeDtypeStruct((m, c), _F32),
        compiler_params=pltpu.CompilerParams(
            dimension_semantics=("arbitrary",)),
    )(feats, w, b.reshape(1, c))


def _mm_body(g_ref, x_ref, o_ref):
    o_ref[...] = _dot(g_ref[...], x_ref[...])


def _gmm(g, x, bm=_BM):
    m, k = g.shape
    c = x.shape[1]
    return pl.pallas_call(
        _mm_body,
        grid=(m // bm,),
        in_specs=[pl.BlockSpec((bm, k), lambda i: (i, 0)),
                  pl.BlockSpec((k, c), lambda i: (0, 0))],
        out_specs=pl.BlockSpec((bm, c), lambda i: (i, 0)),
        out_shape=jax.ShapeDtypeStruct((m, c), _F32),
        compiler_params=pltpu.CompilerParams(
            dimension_semantics=("arbitrary",)),
    )(g, x)


def _pass_u_body(g_ref, imf_ref, tf_ref, iid_ref, tid_ref, iemb_ref, wsum_ref,
                 uid_ref, tuid_ref, uemb_ref,
                 ouf_ref, otf_ref, oug1_ref, oug0_ref, oig0_ref, ig0_scr):
    i = pl.program_id(0)
    bm = g_ref.shape[0]

    @pl.when(i == 0)
    def _():
        ig0_scr[...] = _id_update(iemb_ref[...], iid_ref[...], tid_ref[...],
                                  wsum_ref[...])

    g = g_ref[...]
    ouf_ref[...] = _dot(g, imf_ref[...])
    otf_ref[...] = _dot(g, tf_ref[...])
    oug1_ref[...] = _dot(g, ig0_scr[...])
    oug0_ref[...] = _id_update(uemb_ref[...], uid_ref[...], tuid_ref[...],
                               wsum_ref[...])
    oig0_ref[...] = ig0_scr[pl.ds(i * bm, bm), :]


def _pass_u(ui, image_f, text_f, image_item_id, text_item_id, item_id_emb,
            w_sum, image_user_id, text_user_id, user_id_emb, bm=_BM):
    m, k = ui.shape
    c = _EMBED
    blk = pl.BlockSpec((bm, c), lambda i: (i, 0))
    full = pl.BlockSpec((k, c), lambda i: (0, 0))
    out_sds = jax.ShapeDtypeStruct((m, c), _F32)
    return pl.pallas_call(
        _pass_u_body,
        grid=(m // bm,),
        in_specs=[pl.BlockSpec((bm, k), lambda i: (i, 0)),
                  full, full, full, full, full,
                  pl.BlockSpec((c, c), lambda i: (0, 0)),
                  blk, blk, blk],
        out_specs=[blk] * 5,
        out_shape=[out_sds] * 5,
        scratch_shapes=[pltpu.VMEM((k, c), _F32)],
        compiler_params=pltpu.CompilerParams(
            dimension_semantics=("arbitrary",)),
    )(ui, image_f, text_f, image_item_id, text_item_id, item_id_emb, w_sum,
      image_user_id, text_user_id, user_id_emb)


def _pass_i_body(g_ref, x1_ref, x2_ref, x3_ref, o1_ref, o2_ref, o3_ref):
    g = g_ref[...]
    o1_ref[...] = _dot(g, x1_ref[...])
    o2_ref[...] = _dot(g, x2_ref[...])
    o3_ref[...] = _dot(g, x3_ref[...])


def _pass_i(iu, x1, x2, x3, bm=_BM):
    m, k = iu.shape
    c = _EMBED
    blk = pl.BlockSpec((bm, c), lambda i: (i, 0))
    full = pl.BlockSpec((k, c), lambda i: (0, 0))
    out_sds = jax.ShapeDtypeStruct((m, c), _F32)
    return pl.pallas_call(
        _pass_i_body,
        grid=(m // bm,),
        in_specs=[pl.BlockSpec((bm, k), lambda i: (i, 0)), full, full, full],
        out_specs=[blk] * 3,
        out_shape=[out_sds] * 3,
        compiler_params=pltpu.CompilerParams(
            dimension_semantics=("arbitrary",)),
    )(iu, x1, x2, x3)


def _final(g0, g1, g2, fa, fb):
    mean_g = (g0 + g1 + g2) * (1.0 / 3.0)
    return (mean_g + _MODEL_CAT_RATE * _row_normalize(fa)
            + _MODEL_CAT_RATE * _row_normalize(fb))


def _pass_us_body(g_ref, ig1_ref, ug0_ref, ug1_ref, fu1_ref, fu2_ref,
                  oug2_ref, oug_ref):
    sm = jax.nn.softmax(_dot(g_ref[...], ig1_ref[...]), axis=-1)
    oug2_ref[...] = sm
    oug_ref[...] = _final(ug0_ref[...], ug1_ref[...], sm,
                          fu1_ref[...], fu2_ref[...])


def _pass_us(ui, i_g1, u_g0, u_g1, fu1, fu2, bm=_BM):
    m, k = ui.shape
    c = _EMBED
    blk = pl.BlockSpec((bm, c), lambda i: (i, 0))
    full = pl.BlockSpec((k, c), lambda i: (0, 0))
    out_sds = jax.ShapeDtypeStruct((m, c), _F32)
    return pl.pallas_call(
        _pass_us_body,
        grid=(m // bm,),
        in_specs=[pl.BlockSpec((bm, k), lambda i: (i, 0)),
                  full, blk, blk, blk, blk],
        out_specs=[blk, blk],
        out_shape=[out_sds, out_sds],
        compiler_params=pltpu.CompilerParams(
            dimension_semantics=("arbitrary",)),
    )(ui, i_g1, u_g0, u_g1, fu1, fu2)


def _pass_is_body(g_ref, ug2_ref, ig0_ref, ig1_ref, fi1_ref, fi2_ref,
                  oig_ref):
    sm = jax.nn.softmax(_dot(g_ref[...], ug2_ref[...]), axis=-1)
    oig_ref[...] = _final(ig0_ref[...], ig1_ref[...], sm,
                          fi1_ref[...], fi2_ref[...])


def _pass_is(iu, u_g2, i_g0, i_g1, fi1, fi2, bm=_BM):
    m, k = iu.shape
    c = _EMBED
    blk = pl.BlockSpec((bm, c), lambda i: (i, 0))
    full = pl.BlockSpec((k, c), lambda i: (0, 0))
    return pl.pallas_call(
        _pass_is_body,
        grid=(m // bm,),
        in_specs=[pl.BlockSpec((bm, k), lambda i: (i, 0)),
                  full, blk, blk, blk, blk],
        out_specs=blk,
        out_shape=jax.ShapeDtypeStruct((m, c), _F32),
        compiler_params=pltpu.CompilerParams(
            dimension_semantics=("arbitrary",)),
    )(iu, u_g2, i_g0, i_g1, fi1, fi2)


def kernel(ui_graph, iu_graph, image_ui_graph, image_iu_graph, text_ui_graph,
           text_iu_graph, image_feats, text_feats, w_image_trans, b_image_trans,
           w_text_trans, b_text_trans, user_id_emb, item_id_emb, w_q, w_k, w_cat):
    image_f = _proj(image_feats, w_image_trans, b_image_trans)
    text_f = _proj(text_feats, w_text_trans, b_text_trans)

    image_user_id = _gmm(image_ui_graph, item_id_emb)
    text_user_id = _gmm(text_ui_graph, item_id_emb)
    image_item_id = _gmm(image_iu_graph, user_id_emb)
    text_item_id = _gmm(text_iu_graph, user_id_emb)

    w_sum = w_cat.reshape(_HEADS, _EMBED, _EMBED).sum(0)

    (image_user_feats, text_user_feats, u_g1, u_g0, i_g0) = _pass_u(
        ui_graph, image_f, text_f, image_item_id, text_item_id, item_id_emb,
        w_sum, image_user_id, text_user_id, user_id_emb)

    image_item_feats, text_item_feats, i_g1 = _pass_i(
        iu_graph, image_user_feats, text_user_feats, u_g1)

    u_g2, u_g = _pass_us(ui_graph, i_g1, u_g0, u_g1,
                         image_user_feats, text_user_feats)
    i_g = _pass_is(iu_graph, u_g2, i_g0, i_g1,
                   image_item_feats, text_item_feats)

    return (u_g, i_g, image_item_feats, text_item_feats, image_user_feats,
            text_user_feats, u_g, i_g, image_user_id, text_user_id,
            image_item_id, text_item_id)
